# baseline (speedup 1.0000x reference)
import functools

import jax
import jax.numpy as jnp
from jax import lax
from jax.experimental import pallas as pl
from jax.experimental.pallas import tpu as pltpu
from jax.experimental.pallas import tpu_sc as plsc

NC, NS, L = 2, 16, 16
ROWS, N = 64, 100000
ROWS_PER_SC = ROWS // NC
CHUNK = 6272
NPAD = NS * CHUNK
TAIL = NPAD - N
REAL_LAST = CHUNK - TAIL
NV = CHUNK // L
NQ = 4
NVQ = NV // NQ
QBASE = NVQ * L
NBINS = 256
INT_MIN = jnp.int32(-2**31)

_mesh = None


def _get_mesh():
    global _mesh
    if _mesh is None:
        _mesh = plsc.VectorSubcoreMesh(
            core_axis_name="c", subcore_axis_name="s",
            num_cores=NC, num_subcores=NS)
    return _mesh


def _body(x_hbm, out_hbm, keys_sp, perm_a, perm_b, totals_sp,
          fbuf, kbuf, pbuf, posbuf, hsum, ttbuf, *histoffs):
    hists = histoffs[:NQ]
    offss = histoffs[NQ:]
    c = lax.axis_index("c")
    t = lax.axis_index("s")
    lane = lax.iota(jnp.int32, L)
    zeros16 = jnp.zeros((L,), jnp.int32)
    my_lo = t * CHUNK

    def digits(k, shift):
        return lax.shift_right_logical(k, shift) & 255

    def radix_pass(shift, perm_in, perm_out, first):
        if not first:
            pltpu.sync_copy(perm_in.at[pl.ds(my_lo, CHUNK)], pbuf)
            pltpu.sync_copy(keys_sp.at[pbuf], kbuf)

        for j in range(NBINS // L):
            for q in range(NQ):
                hists[q][pl.ds(L * j, L)] = zeros16

        def hist_body(i, _):
            for q in range(NQ):
                d = digits(kbuf[pl.ds(QBASE * q + L * i, L)], shift)
                cnt, last = plsc.scan_count(d)
                plsc.addupdate_scatter(hists[q], [d], cnt, mask=last)
            return 0
        lax.fori_loop(0, NVQ, hist_body, 0)

        for j in range(NBINS // L):
            sl = pl.ds(L * j, L)
            acc = hists[0][sl]
            for q in range(1, NQ):
                acc = acc + hists[q][sl]
            hsum[sl] = acc
        pltpu.sync_copy(hsum, totals_sp.at[t])
        plsc.subcore_barrier()
        pltpu.sync_copy(totals_sp, ttbuf)

        carry = jnp.int32(0)
        for j in range(NBINS // L):
            sl = pl.ds(L * j, L)

            def tot_body(tp, acc):
                return acc + ttbuf[tp, sl]
            tot = lax.fori_loop(0, NS, tot_body, zeros16)
            below = lax.fori_loop(0, t, tot_body, zeros16)
            base = plsc.cumsum(tot) - tot + below + carry
            for q in range(NQ):
                offss[q][sl] = base
                base = base + hists[q][sl]
            carry = carry + jnp.sum(tot)

        def pos_body(i, _):
            for q in range(NQ):
                d = digits(kbuf[pl.ds(QBASE * q + L * i, L)], shift)
                cnt, last = plsc.scan_count(d)
                off = plsc.load_gather(offss[q], [d])
                posbuf[pl.ds(QBASE * q + L * i, L)] = off + cnt - 1
                plsc.addupdate_scatter(offss[q], [d], cnt, mask=last)
            return 0
        lax.fori_loop(0, NVQ, pos_body, 0)

        pltpu.sync_copy(pbuf, perm_out.at[posbuf])
        plsc.subcore_barrier()

    def row_body(r, _):
        row = c * ROWS_PER_SC + r

        @pl.when(t < NS - 1)
        def _():
            pltpu.sync_copy(x_hbm.at[row, pl.ds(my_lo, CHUNK)], fbuf)

        @pl.when(t == NS - 1)
        def _():
            pltpu.sync_copy(x_hbm.at[row, pl.ds((NS - 1) * CHUNK, REAL_LAST)],
                            fbuf.at[pl.ds(0, REAL_LAST)])

        def xform_body(i, _):
            b = plsc.bitcast(fbuf[pl.ds(L * i, L)], jnp.int32)
            key = b ^ (lax.shift_right_arithmetic(b, 31) | INT_MIN)
            kbuf[pl.ds(L * i, L)] = key
            pbuf[pl.ds(L * i, L)] = my_lo + L * i + lane
            return 0
        lax.fori_loop(0, NV, xform_body, 0)

        @pl.when(t == NS - 1)
        def _():
            for j in range(TAIL // L):
                kbuf[pl.ds(REAL_LAST + L * j, L)] = zeros16 - 1

        pltpu.sync_copy(kbuf, keys_sp.at[pl.ds(my_lo, CHUNK)])

        radix_pass(0, perm_a, perm_b, first=True)
        radix_pass(8, perm_b, perm_a, first=False)
        radix_pass(16, perm_a, perm_b, first=False)
        radix_pass(24, perm_b, perm_a, first=False)

        @pl.when(t < NS - 1)
        def _():
            pltpu.sync_copy(perm_a.at[pl.ds(my_lo, CHUNK)],
                            out_hbm.at[row, pl.ds(my_lo, CHUNK)])

        @pl.when(t == NS - 1)
        def _():
            pltpu.sync_copy(perm_a.at[pl.ds((NS - 1) * CHUNK, REAL_LAST)],
                            out_hbm.at[row, pl.ds((NS - 1) * CHUNK, REAL_LAST)])
        return 0

    lax.fori_loop(0, ROWS_PER_SC, row_body, 0)


@jax.jit
def kernel(x):
    run = functools.partial(
        pl.kernel,
        out_type=jax.ShapeDtypeStruct((ROWS, N), jnp.int32),
        mesh=_get_mesh(),
        scratch_types=[
            pltpu.VMEM_SHARED((NPAD,), jnp.int32),
            pltpu.VMEM_SHARED((NPAD,), jnp.int32),
            pltpu.VMEM_SHARED((NPAD,), jnp.int32),
            pltpu.VMEM_SHARED((NS, NBINS), jnp.int32),
            pltpu.VMEM((CHUNK,), jnp.float32),
            pltpu.VMEM((CHUNK,), jnp.int32),
            pltpu.VMEM((CHUNK,), jnp.int32),
            pltpu.VMEM((CHUNK,), jnp.int32),
            pltpu.VMEM((NBINS,), jnp.int32),
            pltpu.VMEM((NS, NBINS), jnp.int32),
        ] + [pltpu.VMEM((NBINS,), jnp.int32) for _ in range(2 * NQ)],
        compiler_params=pltpu.CompilerParams(
            needs_layout_passes=False, use_tc_tiling_on_sc=False),
    )(_body)
    return run(x)


# speedup vs baseline: 1.4400x; 1.4400x over previous
import functools

import jax
import jax.numpy as jnp
from jax import lax
from jax.experimental import pallas as pl
from jax.experimental.pallas import tpu as pltpu
from jax.experimental.pallas import tpu_sc as plsc

NC, NS, L = 2, 16, 16
ROWS, N = 64, 100000
ROWS_PER_SC = ROWS // NC
CHUNK = 6272
NPAD = NS * CHUNK
TAIL = NPAD - N
REAL_LAST = CHUNK - TAIL
NV = CHUNK // L
NVH = NV // 2
HBASE = NVH * L
NBINS = 256
INT_MIN = jnp.int32(-2**31)

_mesh = None


def _get_mesh():
    global _mesh
    if _mesh is None:
        _mesh = plsc.VectorSubcoreMesh(
            core_axis_name="c", subcore_axis_name="s",
            num_cores=NC, num_subcores=NS)
    return _mesh


def _body(x_hbm, out_hbm, keys_sp, perm_a, perm_b, totals_sp,
          fbuf, kbuf, pbuf, posbuf, hist0, hist1, hsum, offsa, offsb, ttbuf):
    c = lax.axis_index("c")
    t = lax.axis_index("s")
    lane = lax.iota(jnp.int32, L)
    zeros16 = jnp.zeros((L,), jnp.int32)
    my_lo = t * CHUNK

    def digits(k, shift):
        return lax.shift_right_logical(k, shift) & 255

    def radix_pass(shift, perm_in, perm_out, first):
        if not first:
            with jax.named_scope("perm_in"):
                pltpu.sync_copy(perm_in.at[pl.ds(my_lo, CHUNK)], pbuf)
            with jax.named_scope("keys_gather"):
                pltpu.sync_copy(keys_sp.at[pbuf], kbuf)

        _sA = jax.named_scope("histA"); _sA.__enter__()
        for j in range(NBINS // L):
            hist0[pl.ds(L * j, L)] = zeros16
            hist1[pl.ds(L * j, L)] = zeros16

        def hist_body(i, _):
            d0 = digits(kbuf[pl.ds(L * i, L)], shift)
            d1 = digits(kbuf[pl.ds(HBASE + L * i, L)], shift)
            c0, l0 = plsc.scan_count(d0)
            c1, l1 = plsc.scan_count(d1)
            plsc.addupdate_scatter(hist0, [d0], c0, mask=l0)
            plsc.addupdate_scatter(hist1, [d1], c1, mask=l1)
            return 0
        lax.fori_loop(0, NVH, hist_body, 0)
        _sA.__exit__(None, None, None)

        with jax.named_scope("exch"):
            for j in range(NBINS // L):
                sl = pl.ds(L * j, L)
                hsum[sl] = hist0[sl] + hist1[sl]
            pltpu.sync_copy(hsum, totals_sp.at[t])
            plsc.subcore_barrier()
            pltpu.sync_copy(totals_sp, ttbuf)

        _sB = jax.named_scope("phaseB"); _sB.__enter__()
        carry = jnp.int32(0)
        for j in range(NBINS // L):
            sl = pl.ds(L * j, L)

            def tot_body(tp, acc):
                return acc + ttbuf[tp, sl]
            tot = lax.fori_loop(0, NS, tot_body, zeros16)
            below = lax.fori_loop(0, t, tot_body, zeros16)
            excl = plsc.cumsum(tot) - tot
            offsa[sl] = excl + below + carry
            offsb[sl] = excl + below + carry + hist0[sl]
            carry = carry + jnp.sum(tot)
        _sB.__exit__(None, None, None)

        def pos_body(i, _):
            d0 = digits(kbuf[pl.ds(L * i, L)], shift)
            d1 = digits(kbuf[pl.ds(HBASE + L * i, L)], shift)
            c0, l0 = plsc.scan_count(d0)
            c1, l1 = plsc.scan_count(d1)
            o0 = plsc.load_gather(offsa, [d0])
            o1 = plsc.load_gather(offsb, [d1])
            posbuf[pl.ds(L * i, L)] = o0 + c0 - 1
            posbuf[pl.ds(HBASE + L * i, L)] = o1 + c1 - 1
            plsc.addupdate_scatter(offsa, [d0], c0, mask=l0)
            plsc.addupdate_scatter(offsb, [d1], c1, mask=l1)
            return 0
        _sC = jax.named_scope("posC"); _sC.__enter__()
        lax.fori_loop(0, NVH, pos_body, 0)
        _sC.__exit__(None, None, None)

        with jax.named_scope("scatter"):
            pltpu.sync_copy(pbuf, perm_out.at[posbuf])
            plsc.subcore_barrier()

    def row_body(r, _):
        row = c * ROWS_PER_SC + r

        @pl.when(t < NS - 1)
        def _():
            pltpu.sync_copy(x_hbm.at[row, pl.ds(my_lo, CHUNK)], fbuf)

        @pl.when(t == NS - 1)
        def _():
            pltpu.sync_copy(x_hbm.at[row, pl.ds((NS - 1) * CHUNK, REAL_LAST)],
                            fbuf.at[pl.ds(0, REAL_LAST)])

        def xform_body(i, _):
            b = plsc.bitcast(fbuf[pl.ds(L * i, L)], jnp.int32)
            key = b ^ (lax.shift_right_arithmetic(b, 31) | INT_MIN)
            kbuf[pl.ds(L * i, L)] = key
            pbuf[pl.ds(L * i, L)] = my_lo + L * i + lane
            return 0
        lax.fori_loop(0, NV, xform_body, 0)

        @pl.when(t == NS - 1)
        def _():
            for j in range(TAIL // L):
                kbuf[pl.ds(REAL_LAST + L * j, L)] = zeros16 - 1

        pltpu.sync_copy(kbuf, keys_sp.at[pl.ds(my_lo, CHUNK)])

        radix_pass(0, perm_a, perm_b, first=True)
        radix_pass(8, perm_b, perm_a, first=False)
        radix_pass(16, perm_a, perm_b, first=False)
        radix_pass(24, perm_b, perm_a, first=False)

        @pl.when(t < NS - 1)
        def _():
            pltpu.sync_copy(perm_a.at[pl.ds(my_lo, CHUNK)],
                            out_hbm.at[row, pl.ds(my_lo, CHUNK)])

        @pl.when(t == NS - 1)
        def _():
            pltpu.sync_copy(perm_a.at[pl.ds((NS - 1) * CHUNK, REAL_LAST)],
                            out_hbm.at[row, pl.ds((NS - 1) * CHUNK, REAL_LAST)])
        return 0

    lax.fori_loop(0, ROWS_PER_SC, row_body, 0)


@jax.jit
def kernel(x):
    run = functools.partial(
        pl.kernel,
        out_type=jax.ShapeDtypeStruct((ROWS, N), jnp.int32),
        mesh=_get_mesh(),
        scratch_types=[
            pltpu.VMEM_SHARED((NPAD,), jnp.int32),
            pltpu.VMEM_SHARED((NPAD,), jnp.int32),
            pltpu.VMEM_SHARED((NPAD,), jnp.int32),
            pltpu.VMEM_SHARED((NS, NBINS), jnp.int32),
            pltpu.VMEM((CHUNK,), jnp.float32),
            pltpu.VMEM((CHUNK,), jnp.int32),
            pltpu.VMEM((CHUNK,), jnp.int32),
            pltpu.VMEM((CHUNK,), jnp.int32),
            pltpu.VMEM((NBINS,), jnp.int32),
            pltpu.VMEM((NBINS,), jnp.int32),
            pltpu.VMEM((NBINS,), jnp.int32),
            pltpu.VMEM((NBINS,), jnp.int32),
            pltpu.VMEM((NBINS,), jnp.int32),
            pltpu.VMEM((NS, NBINS), jnp.int32),
        ],
        compiler_params=pltpu.CompilerParams(
            needs_layout_passes=False, use_tc_tiling_on_sc=False),
    )(_body)
    return run(x)


# speedup vs baseline: 1.5383x; 1.0682x over previous
import functools

import jax
import jax.numpy as jnp
from jax import lax
from jax.experimental import pallas as pl
from jax.experimental.pallas import tpu as pltpu
from jax.experimental.pallas import tpu_sc as plsc

NC, NS, L = 2, 16, 16
ROWS, N = 64, 100000
ROWS_PER_SC = ROWS // NC
CHUNK = 6272
HH = CHUNK // 2
QH = HH // 2
NVQ = QH // L
NPAD = NS * CHUNK
TAIL = NPAD - N
REAL_LAST = CHUNK - TAIL
PAD_LO = REAL_LAST - HH
NBINS = 256
INT_MIN = jnp.int32(-2**31)

_mesh = None


def _get_mesh():
    global _mesh
    if _mesh is None:
        _mesh = plsc.VectorSubcoreMesh(
            core_axis_name="c", subcore_axis_name="s",
            num_cores=NC, num_subcores=NS)
    return _mesh


def _body(x_hbm, out_hbm, keys_sp, perm_a, perm_b, totals_sp,
          fbuf, kbuf0, kbuf1, pbuf0, pbuf1, posbuf0, posbuf1,
          hist0, hist1, hist2, hist3, hsum, offs0, offs1, offs2, offs3,
          ttbuf, sem0, sem1, sem2, sem3):
    c = lax.axis_index("c")
    t = lax.axis_index("s")
    lane = lax.iota(jnp.int32, L)
    zeros16 = jnp.zeros((L,), jnp.int32)
    my_lo = t * CHUNK

    def digits(k, shift):
        return lax.shift_right_logical(k, shift) & 255

    def hist_loop(kb, ha, hb, shift):
        def body(i, _):
            da = digits(kb[pl.ds(L * i, L)], shift)
            db = digits(kb[pl.ds(QH + L * i, L)], shift)
            ca, la = plsc.scan_count(da)
            cb, lb = plsc.scan_count(db)
            plsc.addupdate_scatter(ha, [da], ca, mask=la)
            plsc.addupdate_scatter(hb, [db], cb, mask=lb)
            return 0
        lax.fori_loop(0, NVQ, body, 0)

    def pos_loop(kb, pob, oa, ob, shift):
        def body(i, _):
            da = digits(kb[pl.ds(L * i, L)], shift)
            db = digits(kb[pl.ds(QH + L * i, L)], shift)
            ca, la = plsc.scan_count(da)
            cb, lb = plsc.scan_count(db)
            va = plsc.load_gather(oa, [da])
            vb = plsc.load_gather(ob, [db])
            pob[pl.ds(L * i, L)] = va + ca - 1
            pob[pl.ds(QH + L * i, L)] = vb + cb - 1
            plsc.addupdate_scatter(oa, [da], ca, mask=la)
            plsc.addupdate_scatter(ob, [db], cb, mask=lb)
            return 0
        lax.fori_loop(0, NVQ, body, 0)

    def radix_pass(shift, perm_in, perm_out, first):
        if not first:
            dp0 = pltpu.async_copy(perm_in.at[pl.ds(my_lo, HH)], pbuf0, sem0)
            dp1 = pltpu.async_copy(perm_in.at[pl.ds(my_lo + HH, HH)],
                                   pbuf1, sem1)
            dp0.wait()
            dk0 = pltpu.async_copy(keys_sp.at[pbuf0], kbuf0, sem2)
            dp1.wait()
            dk1 = pltpu.async_copy(keys_sp.at[pbuf1], kbuf1, sem3)
        for j in range(NBINS // L):
            hist0[pl.ds(L * j, L)] = zeros16
            hist1[pl.ds(L * j, L)] = zeros16
            hist2[pl.ds(L * j, L)] = zeros16
            hist3[pl.ds(L * j, L)] = zeros16
        if not first:
            dk0.wait()
        hist_loop(kbuf0, hist0, hist1, shift)
        if not first:
            dk1.wait()
        hist_loop(kbuf1, hist2, hist3, shift)

        for j in range(NBINS // L):
            sl = pl.ds(L * j, L)
            hsum[sl] = (hist0[sl] + hist1[sl]) + (hist2[sl] + hist3[sl])
        pltpu.sync_copy(hsum, totals_sp.at[t])
        plsc.subcore_barrier()
        pltpu.sync_copy(totals_sp, ttbuf)

        carry = jnp.int32(0)
        for j in range(NBINS // L):
            sl = pl.ds(L * j, L)

            def tot_body(tp, acc):
                return acc + ttbuf[tp, sl]
            tot = lax.fori_loop(0, NS, tot_body, zeros16)
            below = lax.fori_loop(0, t, tot_body, zeros16)
            base = plsc.cumsum(tot) - tot + below + carry
            offs0[sl] = base
            base = base + hist0[sl]
            offs1[sl] = base
            base = base + hist1[sl]
            offs2[sl] = base
            base = base + hist2[sl]
            offs3[sl] = base
            carry = carry + jnp.sum(tot)

        pos_loop(kbuf0, posbuf0, offs0, offs1, shift)
        ds0 = pltpu.async_copy(pbuf0, perm_out.at[posbuf0], sem0)
        pos_loop(kbuf1, posbuf1, offs2, offs3, shift)
        ds1 = pltpu.async_copy(pbuf1, perm_out.at[posbuf1], sem1)
        ds0.wait()
        ds1.wait()
        plsc.subcore_barrier()

    def row_body(r, _):
        row = c * ROWS_PER_SC + r

        @pl.when(t < NS - 1)
        def _():
            pltpu.sync_copy(x_hbm.at[row, pl.ds(my_lo, CHUNK)], fbuf)

        @pl.when(t == NS - 1)
        def _():
            pltpu.sync_copy(x_hbm.at[row, pl.ds((NS - 1) * CHUNK, REAL_LAST)],
                            fbuf.at[pl.ds(0, REAL_LAST)])

        def xform(fb_off, kb, pb):
            def body(i, _):
                b = plsc.bitcast(fbuf[pl.ds(fb_off + L * i, L)], jnp.int32)
                key = b ^ (lax.shift_right_arithmetic(b, 31) | INT_MIN)
                kb[pl.ds(L * i, L)] = key
                pb[pl.ds(L * i, L)] = my_lo + fb_off + L * i + lane
                return 0
            lax.fori_loop(0, HH // L, body, 0)
        xform(0, kbuf0, pbuf0)
        xform(HH, kbuf1, pbuf1)

        @pl.when(t == NS - 1)
        def _():
            for j in range(TAIL // L):
                kbuf1[pl.ds(PAD_LO + L * j, L)] = zeros16 - 1

        dk0 = pltpu.async_copy(kbuf0, keys_sp.at[pl.ds(my_lo, HH)], sem2)
        dk1 = pltpu.async_copy(kbuf1, keys_sp.at[pl.ds(my_lo + HH, HH)], sem3)
        dk0.wait()
        dk1.wait()

        radix_pass(0, perm_a, perm_b, first=True)
        radix_pass(8, perm_b, perm_a, first=False)
        radix_pass(16, perm_a, perm_b, first=False)
        radix_pass(24, perm_b, perm_a, first=False)

        @pl.when(t < NS - 1)
        def _():
            pltpu.sync_copy(perm_a.at[pl.ds(my_lo, CHUNK)],
                            out_hbm.at[row, pl.ds(my_lo, CHUNK)])

        @pl.when(t == NS - 1)
        def _():
            pltpu.sync_copy(perm_a.at[pl.ds((NS - 1) * CHUNK, REAL_LAST)],
                            out_hbm.at[row, pl.ds((NS - 1) * CHUNK, REAL_LAST)])
        return 0

    lax.fori_loop(0, ROWS_PER_SC, row_body, 0)


@jax.jit
def kernel(x):
    run = functools.partial(
        pl.kernel,
        out_type=jax.ShapeDtypeStruct((ROWS, N), jnp.int32),
        mesh=_get_mesh(),
        scratch_types=[
            pltpu.VMEM_SHARED((NPAD,), jnp.int32),
            pltpu.VMEM_SHARED((NPAD,), jnp.int32),
            pltpu.VMEM_SHARED((NPAD,), jnp.int32),
            pltpu.VMEM_SHARED((NS, NBINS), jnp.int32),
            pltpu.VMEM((CHUNK,), jnp.float32),
            pltpu.VMEM((HH,), jnp.int32),
            pltpu.VMEM((HH,), jnp.int32),
            pltpu.VMEM((HH,), jnp.int32),
            pltpu.VMEM((HH,), jnp.int32),
            pltpu.VMEM((HH,), jnp.int32),
            pltpu.VMEM((HH,), jnp.int32),
            pltpu.VMEM((NBINS,), jnp.int32),
            pltpu.VMEM((NBINS,), jnp.int32),
            pltpu.VMEM((NBINS,), jnp.int32),
            pltpu.VMEM((NBINS,), jnp.int32),
            pltpu.VMEM((NBINS,), jnp.int32),
            pltpu.VMEM((NBINS,), jnp.int32),
            pltpu.VMEM((NBINS,), jnp.int32),
            pltpu.VMEM((NBINS,), jnp.int32),
            pltpu.VMEM((NBINS,), jnp.int32),
            pltpu.VMEM((NS, NBINS), jnp.int32),
            pltpu.SemaphoreType.DMA,
            pltpu.SemaphoreType.DMA,
            pltpu.SemaphoreType.DMA,
            pltpu.SemaphoreType.DMA,
        ],
        compiler_params=pltpu.CompilerParams(
            needs_layout_passes=False, use_tc_tiling_on_sc=False),
    )(_body)
    return run(x)


# speedup vs baseline: 1.8205x; 1.1835x over previous
import functools

import jax
import jax.numpy as jnp
from jax import lax
from jax.experimental import pallas as pl
from jax.experimental.pallas import tpu as pltpu
from jax.experimental.pallas import tpu_sc as plsc

NC, NS, L = 2, 16, 16
ROWS, N = 64, 100000
ROWS_PER_SC = ROWS // NC
CHUNK = 6272
HH = CHUNK // 2
QH = HH // 2
NVQ = QH // L
NPAD = NS * CHUNK
TAIL = NPAD - N
REAL_LAST = CHUNK - TAIL
PAD_LO = REAL_LAST - HH
NBINS = 256
INT_MIN = jnp.int32(-2**31)

_mesh = None


def _get_mesh():
    global _mesh
    if _mesh is None:
        _mesh = plsc.VectorSubcoreMesh(
            core_axis_name="c", subcore_axis_name="s",
            num_cores=NC, num_subcores=NS)
    return _mesh


def _body(x_hbm, out_hbm, keys_sp, perm_a, perm_b, totals_sp,
          fbuf, kbuf0, kbuf1, pbuf0, pbuf1, posbuf0, posbuf1,
          hist0, hist1, hist2, hist3, hsum, offs0, offs1, offs2, offs3,
          ttbuf, sem0, sem1, sem2, sem3):
    c = lax.axis_index("c")
    t = lax.axis_index("s")
    lane = lax.iota(jnp.int32, L)
    zeros16 = jnp.zeros((L,), jnp.int32)
    my_lo = t * CHUNK

    def digits(k, shift):
        return lax.shift_right_logical(k, shift) & 255

    def scans(kb, i, shift):
        da = digits(kb[pl.ds(L * i, L)], shift)
        db = digits(kb[pl.ds(QH + L * i, L)], shift)
        ca, la = plsc.scan_count(da)
        cb, lb = plsc.scan_count(db)
        return da, ca, la, db, cb, lb

    def hist_loop(kb, ha, hb, shift):
        def body(i, car):
            da, ca, la, db, cb, lb = car
            nxt = scans(kb, i + 1, shift)
            plsc.addupdate_scatter(ha, [da], ca, mask=la)
            plsc.addupdate_scatter(hb, [db], cb, mask=lb)
            return nxt
        lax.fori_loop(0, NVQ, body, scans(kb, 0, shift))

    def pos_loop(kb, pob, oa, ob, shift):
        def body(i, car):
            da, ca, la, db, cb, lb = car
            nxt = scans(kb, i + 1, shift)
            va = plsc.load_gather(oa, [da])
            vb = plsc.load_gather(ob, [db])
            pob[pl.ds(L * i, L)] = va + ca - 1
            pob[pl.ds(QH + L * i, L)] = vb + cb - 1
            plsc.addupdate_scatter(oa, [da], ca, mask=la)
            plsc.addupdate_scatter(ob, [db], cb, mask=lb)
            return nxt
        lax.fori_loop(0, NVQ, body, scans(kb, 0, shift))

    def radix_pass(shift, perm_in, perm_out, first):
        if not first:
            dp0 = pltpu.async_copy(perm_in.at[pl.ds(my_lo, HH)], pbuf0, sem0)
            dp1 = pltpu.async_copy(perm_in.at[pl.ds(my_lo + HH, HH)],
                                   pbuf1, sem1)
            dp0.wait()
            dk0 = pltpu.async_copy(keys_sp.at[pbuf0],
                                   kbuf0.at[pl.ds(0, HH)], sem2)
            dp1.wait()
            dk1 = pltpu.async_copy(keys_sp.at[pbuf1],
                                   kbuf1.at[pl.ds(0, HH)], sem3)
        for j in range(NBINS // L):
            hist0[pl.ds(L * j, L)] = zeros16
            hist1[pl.ds(L * j, L)] = zeros16
            hist2[pl.ds(L * j, L)] = zeros16
            hist3[pl.ds(L * j, L)] = zeros16
        if not first:
            dk0.wait()
        hist_loop(kbuf0, hist0, hist1, shift)
        if not first:
            dk1.wait()
        hist_loop(kbuf1, hist2, hist3, shift)

        for j in range(NBINS // L):
            sl = pl.ds(L * j, L)
            hsum[sl] = (hist0[sl] + hist1[sl]) + (hist2[sl] + hist3[sl])
        pltpu.sync_copy(hsum, totals_sp.at[t])
        plsc.subcore_barrier()
        pltpu.sync_copy(totals_sp, ttbuf)

        carry = jnp.int32(0)
        for j in range(NBINS // L):
            sl = pl.ds(L * j, L)

            def tot_body(tp, acc):
                return acc + ttbuf[tp, sl]
            tot = lax.fori_loop(0, NS, tot_body, zeros16)
            below = lax.fori_loop(0, t, tot_body, zeros16)
            base = plsc.cumsum(tot) - tot + below + carry
            offs0[sl] = base
            base = base + hist0[sl]
            offs1[sl] = base
            base = base + hist1[sl]
            offs2[sl] = base
            base = base + hist2[sl]
            offs3[sl] = base
            carry = carry + jnp.sum(tot)

        pos_loop(kbuf0, posbuf0, offs0, offs1, shift)
        ds0 = pltpu.async_copy(pbuf0, perm_out.at[posbuf0], sem0)
        pos_loop(kbuf1, posbuf1, offs2, offs3, shift)
        ds1 = pltpu.async_copy(pbuf1, perm_out.at[posbuf1], sem1)
        ds0.wait()
        ds1.wait()
        plsc.subcore_barrier()

    def row_body(r, _):
        row = c * ROWS_PER_SC + r

        @pl.when(t < NS - 1)
        def _():
            pltpu.sync_copy(x_hbm.at[row, pl.ds(my_lo, CHUNK)], fbuf)

        @pl.when(t == NS - 1)
        def _():
            pltpu.sync_copy(x_hbm.at[row, pl.ds((NS - 1) * CHUNK, REAL_LAST)],
                            fbuf.at[pl.ds(0, REAL_LAST)])

        def xform(fb_off, kb, pb):
            def body(i, _):
                b = plsc.bitcast(fbuf[pl.ds(fb_off + L * i, L)], jnp.int32)
                key = b ^ (lax.shift_right_arithmetic(b, 31) | INT_MIN)
                kb[pl.ds(L * i, L)] = key
                pb[pl.ds(L * i, L)] = my_lo + fb_off + L * i + lane
                return 0
            lax.fori_loop(0, HH // L, body, 0)
        xform(0, kbuf0, pbuf0)
        xform(HH, kbuf1, pbuf1)

        @pl.when(t == NS - 1)
        def _():
            for j in range(TAIL // L):
                kbuf1[pl.ds(PAD_LO + L * j, L)] = zeros16 - 1

        dk0 = pltpu.async_copy(kbuf0.at[pl.ds(0, HH)],
                               keys_sp.at[pl.ds(my_lo, HH)], sem2)
        dk1 = pltpu.async_copy(kbuf1.at[pl.ds(0, HH)],
                               keys_sp.at[pl.ds(my_lo + HH, HH)], sem3)
        dk0.wait()
        dk1.wait()

        radix_pass(0, perm_a, perm_b, first=True)
        radix_pass(8, perm_b, perm_a, first=False)
        radix_pass(16, perm_a, perm_b, first=False)
        radix_pass(24, perm_b, perm_a, first=False)

        @pl.when(t < NS - 1)
        def _():
            pltpu.sync_copy(perm_a.at[pl.ds(my_lo, CHUNK)],
                            out_hbm.at[row, pl.ds(my_lo, CHUNK)])

        @pl.when(t == NS - 1)
        def _():
            pltpu.sync_copy(perm_a.at[pl.ds((NS - 1) * CHUNK, REAL_LAST)],
                            out_hbm.at[row, pl.ds((NS - 1) * CHUNK, REAL_LAST)])
        return 0

    lax.fori_loop(0, ROWS_PER_SC, row_body, 0)


@jax.jit
def kernel(x):
    run = functools.partial(
        pl.kernel,
        out_type=jax.ShapeDtypeStruct((ROWS, N), jnp.int32),
        mesh=_get_mesh(),
        scratch_types=[
            pltpu.VMEM_SHARED((NPAD,), jnp.int32),
            pltpu.VMEM_SHARED((NPAD,), jnp.int32),
            pltpu.VMEM_SHARED((NPAD,), jnp.int32),
            pltpu.VMEM_SHARED((NS, NBINS), jnp.int32),
            pltpu.VMEM((CHUNK,), jnp.float32),
            pltpu.VMEM((HH + L,), jnp.int32),
            pltpu.VMEM((HH + L,), jnp.int32),
            pltpu.VMEM((HH,), jnp.int32),
            pltpu.VMEM((HH,), jnp.int32),
            pltpu.VMEM((HH,), jnp.int32),
            pltpu.VMEM((HH,), jnp.int32),
            pltpu.VMEM((NBINS,), jnp.int32),
            pltpu.VMEM((NBINS,), jnp.int32),
            pltpu.VMEM((NBINS,), jnp.int32),
            pltpu.VMEM((NBINS,), jnp.int32),
            pltpu.VMEM((NBINS,), jnp.int32),
            pltpu.VMEM((NBINS,), jnp.int32),
            pltpu.VMEM((NBINS,), jnp.int32),
            pltpu.VMEM((NBINS,), jnp.int32),
            pltpu.VMEM((NBINS,), jnp.int32),
            pltpu.VMEM((NS, NBINS), jnp.int32),
            pltpu.SemaphoreType.DMA,
            pltpu.SemaphoreType.DMA,
            pltpu.SemaphoreType.DMA,
            pltpu.SemaphoreType.DMA,
        ],
        compiler_params=pltpu.CompilerParams(
            needs_layout_passes=False, use_tc_tiling_on_sc=False),
    )(_body)
    return run(x)
